# Initial kernel scaffold; baseline (speedup 1.0000x reference)
import jax
import jax.numpy as jnp
from jax.experimental import pallas as pl


def kernel(x, params):
    raise NotImplementedError("write your pallas kernel here")


# speedup vs baseline: 11.3932x; 11.3932x over previous
import functools

import jax
import jax.numpy as jnp
from jax.experimental import pallas as pl

N_PIX = 3072
DEPTH = 6
K_CHEB = 3

_SIZES = []
_n = N_PIX
for _ in range(DEPTH):
    _SIZES.append(_n)
    _n //= 4
_SIZES = _SIZES[::-1]


def _lap_stencil(v, n):
    acc = None
    for o in (1, 2, 3, 4):
        for s in (o % n, (n - o % n) % n):
            if s == 0:
                t = v
            else:
                t = jnp.concatenate([v[s:], v[:s]], axis=0)
            acc = t if acc is None else acc + t
    return acc * (-0.125)


def _cheb_feats(v, n):
    v1 = _lap_stencil(v, n)
    v2 = 2.0 * _lap_stencil(v1, n) - v
    return jnp.concatenate([v, v1, v2], axis=1)


def _lstm_layer_kernel(cur_ref, wx_ref, wh_ref, b_ref, out_ref, *, n, chid):
    wx = wx_ref[:]
    b = b_ref[:]
    a0 = jnp.dot(_cheb_feats(cur_ref[0], n), wx,
                 preferred_element_type=jnp.float32)
    a1 = jnp.dot(_cheb_feats(cur_ref[1], n), wx,
                 preferred_element_type=jnp.float32)

    def act(gates, c_prev):
        i = jax.nn.sigmoid(gates[:, :chid])
        f = jax.nn.sigmoid(gates[:, chid:2 * chid])
        o = jax.nn.sigmoid(gates[:, 2 * chid:3 * chid])
        g = jnp.tanh(gates[:, 3 * chid:])
        c = f * c_prev + i * g
        return o * jnp.tanh(c), c

    h0, c0 = act(a0 + b, jnp.zeros((n, chid), jnp.float32))
    g1 = a1 + jnp.dot(_cheb_feats(h0, n), wh_ref[:],
                      preferred_element_type=jnp.float32) + b
    h1, _ = act(g1, c0)
    out_ref[0] = h0
    out_ref[1] = h1


def _lstm_layer(cur, W, bvec, chid):
    _, n, cin = cur.shape
    wx = W[:, :cin, :].reshape(K_CHEB * cin, 4 * chid)
    wh = W[:, cin:, :].reshape(K_CHEB * chid, 4 * chid)
    return pl.pallas_call(
        functools.partial(_lstm_layer_kernel, n=n, chid=chid),
        out_shape=jax.ShapeDtypeStruct((2, n, chid), jnp.float32),
    )(cur, wx, wh, bvec.reshape(1, 4 * chid))


def _convlstm_stage(x4, Ws, bs, chid):
    cur = jnp.transpose(x4[0], (0, 2, 1))
    for W, bvec in zip(Ws, bs):
        cur = _lstm_layer(cur, W, bvec, chid)
    return jnp.transpose(cur, (0, 2, 1))[None]


def _bn_kernel(x_ref, g_ref, be_ref, o_ref, *, pool):
    xx = x_ref[:]
    mu = jnp.mean(xx, axis=0, keepdims=True)
    var = jnp.mean((xx - mu) * (xx - mu), axis=0, keepdims=True)
    xb = (xx - mu) * jax.lax.rsqrt(var + 1e-5) * g_ref[:] + be_ref[:]
    xb = jnp.maximum(xb, 0.0)
    if pool:
        c = xb.shape[1]
        rows = jax.lax.broadcasted_iota(jnp.int32, (c, c // 4), 0)
        cols = jax.lax.broadcasted_iota(jnp.int32, (c, c // 4), 1)
        pmat = jnp.where(rows // 4 == cols, 0.25, 0.0).astype(jnp.float32)
        o_ref[:] = jnp.dot(xb, pmat, precision=jax.lax.Precision.HIGHEST,
                           preferred_element_type=jnp.float32)
    else:
        o_ref[:] = xb


def _post_bn_relu(x4, g, be, pool):
    d1, d2, d3, n = x4.shape
    x2d = x4.reshape(-1, d3)
    out_c = d3 // 4 if pool else d3
    out = pl.pallas_call(
        functools.partial(_bn_kernel, pool=pool),
        out_shape=jax.ShapeDtypeStruct((x2d.shape[0], out_c), jnp.float32),
    )(x2d, g.reshape(1, d3), be.reshape(1, d3))
    return out, (d1, d2, d3)


def kernel(x, params):
    p = params
    sz = _SIZES

    def enc(x4, name, bn_name, chid, pool):
        x4 = _convlstm_stage(x4, p[name]['W'], p[name]['b'], chid)
        out, (d1, d2, d3) = _post_bn_relu(
            x4, p[bn_name]['g'], p[bn_name]['be'], pool)
        return out, (d1, d2, d3)

    out, (d1, d2, d3) = enc(x, 'enc1', 'bn1', 64, True)
    x_enc5 = out.reshape(-1, sz[4], 1)
    out, (d1, d2, d3) = enc(x_enc5.reshape(d1, d2, d3, -1), 'enc2', 'bn2', 128, True)
    x_enc4 = out.reshape(-1, sz[3], 1)
    out, (d1, d2, d3) = enc(x_enc4.reshape(d1, d2, d3, -1), 'enc3', 'bn3', 256, True)
    x_enc3 = out.reshape(-1, sz[2], 1)
    out, (d1, d2, d3) = enc(x_enc3.reshape(d1, d2, d3, -1), 'enc4', 'bn4', 512, True)
    x_enc2 = out.reshape(-1, sz[1], 1)
    out, (d1, d2, d3) = enc(x_enc2.reshape(d1, d2, d3, -1), 'enc5', 'bn5', 512, True)
    x_enc1 = out.reshape(-1, sz[0], 1)
    out, (d1, d2, d3) = enc(x_enc1.reshape(d1, d2, d3, -1), 'enc6', 'bn6', 512, False)
    x_enc0 = out.reshape(-1, sz[0], 1)

    xcat = jnp.concatenate([x_enc0, x_enc1, x_enc0, x_enc1], axis=1)[:, :sz[1], :]
    out, (d1, d2, d3) = enc(xcat.reshape(d1, d2, d3, -1), 'dec5', 'dbn5', 512, False)
    xcat = jnp.concatenate([out.reshape(-1, sz[1], 1), x_enc2] * 2, axis=1)[:, :sz[2], :]
    out, (d1, d2, d3) = enc(xcat.reshape(d1, d2, d3, -1), 'dec4', 'dbn4', 256, False)
    xcat = jnp.concatenate([out.reshape(-1, sz[2], 1), x_enc3] * 2, axis=1)[:, :sz[3], :]
    out, (d1, d2, d3) = enc(xcat.reshape(d1, d2, d3, -1), 'dec3', 'dbn3', 128, False)
    xcat = jnp.concatenate([out.reshape(-1, sz[3], 1), x_enc4] * 2, axis=1)[:, :sz[4], :]
    out, (d1, d2, d3) = enc(xcat.reshape(d1, d2, d3, -1), 'dec2', 'dbn2', 64, False)
    xup = jnp.repeat(out.reshape(-1, sz[4], 1), 4, axis=1)
    out, (d1, d2, d3) = enc(xup.reshape(d1, d2, d3, -1), 'dec1', 'dbn1', 3, False)
    xr = out.reshape(d1, d2, d3, -1)
    return xr[:, -1:, :, :]
